# Initial kernel scaffold; baseline (speedup 1.0000x reference)
import jax
import jax.numpy as jnp
from jax.experimental import pallas as pl


def kernel(x, edge_index, batch, W1, b1, W0, b0, Wl1, bl1, W2, b2):
    raise NotImplementedError("write your pallas kernel here")


# speedup vs baseline: 9.0507x; 9.0507x over previous
import functools

import jax
import jax.numpy as jnp
from jax import lax
from jax.experimental import pallas as pl
from jax.experimental.pallas import tpu as pltpu
from jax.experimental.pallas import tpu_sc as plsc

N = 10000
NPAD = 10240
E = 320000
D = 128
NG = 128
NCLS = 2

TWO_E = 2 * E
G_TOT = TWO_E + N
NC, NS = 2, 16
NSUB = NC * NS
ROWS = -(-G_TOT // (NSUB * 128))
CH = ROWS * 128
GP = NSUB * CH
DMA_GROUP = 8


def _wid():
    return lax.axis_index("s") * NC + lax.axis_index("c")


def _t0_codes(src_ref, dst_ref, code_ref, tick_ref):
    s = src_ref[...]
    d = dst_ref[...]
    nrows = GP // 128
    t = (lax.broadcasted_iota(jnp.int32, (nrows, 128), 0) * 128
         + lax.broadcasted_iota(jnp.int32, (nrows, 128), 1))
    code_ref[...] = d * NPAD + s
    tick_ref[...] = jnp.where(s != d, t + 1, TWO_E + 1 + s)


def _s1_scatter(codes3, ticks3, slot, idx_v, val_v, sem):
    wid = _wid()
    pltpu.sync_copy(codes3.at[wid], idx_v)
    pltpu.sync_copy(ticks3.at[wid], val_v)
    for g in range(0, ROWS, DMA_GROUP):
        descs = [
            pltpu.async_copy(val_v.at[r], slot.at[idx_v.at[r]], sem)
            for r in range(g, min(g + DMA_GROUP, ROWS))
        ]
        for dsc in descs:
            dsc.wait()


def _s2_degree(codes_f, dstd, slot, parts, idx_v, dst_v, got_v, deg_v, sem):
    wid = _wid()
    base = wid * CH
    pltpu.sync_copy(codes_f.at[pl.ds(base, CH)], idx_v)
    pltpu.sync_copy(dstd.at[pl.ds(base, CH)], dst_v)
    for g in range(0, ROWS, DMA_GROUP):
        descs = [
            pltpu.async_copy(slot.at[idx_v.at[pl.ds(r * 128, 128)]],
                             got_v.at[pl.ds(r * 128, 128)], sem)
            for r in range(g, min(g + DMA_GROUP, ROWS))
        ]
        for dsc in descs:
            dsc.wait()

    def zero(i, c):
        deg_v[pl.ds(i * 16, 16)] = jnp.zeros((16,), jnp.float32)
        return c

    lax.fori_loop(0, N // 16, zero, 0)

    def acc(r, c):
        for cc in range(8):
            off = r * 128 + cc * 16
            got = got_v[pl.ds(off, 16)]
            d = dst_v[pl.ds(off, 16)]
            gt = base + off + lax.iota(jnp.int32, 16)
            keep = jnp.where(got == gt + 1, 1.0, 0.0)
            plsc.addupdate_scatter(deg_v, [d], keep)
        return c

    lax.fori_loop(0, ROWS, acc, 0)
    pltpu.sync_copy(deg_v, parts.at[wid])


def _t1_prep(parts_ref, x_ref, w1_ref, g_ref, dinv_ref):
    ones32 = jnp.ones((NSUB, 1), jnp.float32)
    deg_col = lax.dot_general(parts_ref[...], ones32,
                              (((0,), (0,)), ((), ())),
                              preferred_element_type=jnp.float32,
                              precision=lax.Precision.HIGHEST)
    dinv_col = lax.rsqrt(deg_col)
    h = jnp.dot(x_ref[...], w1_ref[...], preferred_element_type=jnp.float32,
                precision=lax.Precision.HIGHEST)
    g_ref[...] = h * dinv_col
    dinv_ref[...] = dinv_col


def _t2_matmul(slot_ref, g_ref, dinv_ref, b1_ref, h1_ref, acc_ref, *, ncol):
    j = pl.program_id(1)

    @pl.when(j == 0)
    def _():
        acc_ref[...] = jnp.zeros_like(acc_ref)

    a = jnp.where(slot_ref[...] != 0, 1.0, 0.0)
    acc_ref[...] += jnp.dot(a, g_ref[...],
                            preferred_element_type=jnp.float32)

    @pl.when(j == ncol - 1)
    def _():
        out = acc_ref[...] * dinv_ref[...] + b1_ref[...]
        h1_ref[...] = jnp.maximum(out, 0.0)


def _t3_head(h1_ref, x_ref, batch_ref, shift_ref, w0_ref, b0_ref, wl1_ref,
             bl1_ref, w2_ref, b2_ref, out_ref):
    hi = lax.Precision.HIGHEST
    batch_col = batch_ref[...]
    shift_col = shift_ref[...]
    gid_row = lax.broadcasted_iota(jnp.int32, (1, NG), 1)
    first_ge = (batch_col >= gid_row) & (shift_col < gid_row)
    node_col = lax.broadcasted_iota(jnp.int32, (N, 1), 0)
    overflow = (node_col == N - 1) & (batch_col < gid_row)
    onehot = jnp.where(first_ge | overflow, 1.0, 0.0)
    xr = lax.dot_general(onehot, x_ref[...], (((0,), (0,)), ((), ())),
                         preferred_element_type=jnp.float32, precision=hi)
    news = jnp.maximum(
        jnp.dot(xr, w0_ref[...], preferred_element_type=jnp.float32,
                precision=hi) + b0_ref[...], 0.0)

    h1 = h1_ref[...]
    gi_col = lax.broadcasted_iota(jnp.int32, (NG, 1), 0)

    def seg(g, hp):
        m2 = jnp.max(jnp.where(batch_col == g, h1, -jnp.inf), axis=0,
                     keepdims=True)
        return jnp.maximum(hp, jnp.where(gi_col == g, m2, -jnp.inf))

    hp = lax.fori_loop(0, NG, seg, jnp.full((NG, D), -jnp.inf,
                                            dtype=jnp.float32))
    cat = jnp.concatenate([news, hp], axis=1)
    h2 = jnp.maximum(
        jnp.dot(cat, wl1_ref[...], preferred_element_type=jnp.float32,
                precision=hi) + bl1_ref[...], 0.0)
    logits = jnp.dot(h2, w2_ref[...], preferred_element_type=jnp.float32,
                     precision=hi) + b2_ref[...]
    mx = jnp.max(logits, axis=1, keepdims=True)
    lse = mx + jnp.log(jnp.sum(jnp.exp(logits - mx), axis=1, keepdims=True))
    out_ref[...] = logits - lse


def kernel(x, edge_index, batch, W1, b1, W0, b0, Wl1, bl1, W2, b2):
    ei0 = edge_index[0].astype(jnp.int32)
    ei1 = edge_index[1].astype(jnp.int32)
    ar = jnp.arange(N, dtype=jnp.int32)
    pad = jnp.zeros((GP - G_TOT,), jnp.int32)
    srcd = jnp.concatenate([ei0, ei1, ar, pad])
    dstd = jnp.concatenate([ei1, ei0, ar, pad])

    nrows_g = GP // 128
    codes_f, ticks_f = pl.pallas_call(
        _t0_codes,
        in_specs=[
            pl.BlockSpec((nrows_g, 128), lambda: (0, 0)),
            pl.BlockSpec((nrows_g, 128), lambda: (0, 0)),
        ],
        out_specs=[
            pl.BlockSpec((nrows_g, 128), lambda: (0, 0)),
            pl.BlockSpec((nrows_g, 128), lambda: (0, 0)),
        ],
        out_shape=[
            jax.ShapeDtypeStruct((nrows_g, 128), jnp.int32),
            jax.ShapeDtypeStruct((nrows_g, 128), jnp.int32),
        ],
    )(srcd.reshape(nrows_g, 128), dstd.reshape(nrows_g, 128))
    codes3 = codes_f.reshape(NSUB, ROWS, 128)
    ticks3 = ticks_f.reshape(NSUB, ROWS, 128)

    mesh = plsc.VectorSubcoreMesh(core_axis_name="c", subcore_axis_name="s")

    slot_ref = jax.new_ref(jnp.zeros((N * NPAD,), jnp.int32))
    pl.kernel(
        _s1_scatter,
        out_type=(),
        mesh=mesh,
        scratch_types=[
            pltpu.VMEM((ROWS, 128), jnp.int32),
            pltpu.VMEM((ROWS, 128), jnp.int32),
            pltpu.SemaphoreType.DMA,
        ],
    )(codes3, ticks3, slot_ref)
    slot = jax.freeze(slot_ref)

    parts = pl.kernel(
        _s2_degree,
        out_type=jax.ShapeDtypeStruct((NSUB, N), jnp.float32),
        mesh=mesh,
        scratch_types=[
            pltpu.VMEM((CH,), jnp.int32),
            pltpu.VMEM((CH,), jnp.int32),
            pltpu.VMEM((CH,), jnp.int32),
            pltpu.VMEM((N,), jnp.float32),
            pltpu.SemaphoreType.DMA,
        ],
        compiler_params=pltpu.CompilerParams(needs_layout_passes=False),
    )(codes_f.reshape(GP), dstd, slot)

    rb = 1000
    nrow = N // rb
    g, dinv = pl.pallas_call(
        _t1_prep,
        in_specs=[
            pl.BlockSpec((NSUB, N), lambda: (0, 0)),
            pl.BlockSpec((N, D), lambda: (0, 0)),
            pl.BlockSpec((D, D), lambda: (0, 0)),
        ],
        out_specs=[
            pl.BlockSpec((N, D), lambda: (0, 0)),
            pl.BlockSpec((N, 1), lambda: (0, 0)),
        ],
        out_shape=[
            jax.ShapeDtypeStruct((N, D), jnp.float32),
            jax.ShapeDtypeStruct((N, 1), jnp.float32),
        ],
    )(parts, x, W1)

    slot2d = slot.reshape(N, NPAD)
    gpad = jnp.concatenate([g, jnp.zeros((NPAD - N, D), jnp.float32)])
    cb = 1024
    ncol = NPAD // cb
    h1 = pl.pallas_call(
        functools.partial(_t2_matmul, ncol=ncol),
        grid=(nrow, ncol),
        in_specs=[
            pl.BlockSpec((rb, cb), lambda i, j: (i, j)),
            pl.BlockSpec((cb, D), lambda i, j: (j, 0)),
            pl.BlockSpec((rb, 1), lambda i, j: (i, 0)),
            pl.BlockSpec((1, D), lambda i, j: (0, 0)),
        ],
        out_specs=pl.BlockSpec((rb, D), lambda i, j: (i, 0)),
        out_shape=jax.ShapeDtypeStruct((N, D), jnp.float32),
        scratch_shapes=[pltpu.VMEM((rb, D), jnp.float32)],
        compiler_params=pltpu.CompilerParams(
            dimension_semantics=("parallel", "arbitrary")),
    )(slot2d, gpad, dinv, b1.reshape(1, D))

    out = pl.pallas_call(
        _t3_head,
        in_specs=[
            pl.BlockSpec((N, D), lambda: (0, 0)),
            pl.BlockSpec((N, D), lambda: (0, 0)),
            pl.BlockSpec((N, 1), lambda: (0, 0)),
            pl.BlockSpec((N, 1), lambda: (0, 0)),
            pl.BlockSpec((D, D), lambda: (0, 0)),
            pl.BlockSpec((1, D), lambda: (0, 0)),
            pl.BlockSpec((2 * D, D), lambda: (0, 0)),
            pl.BlockSpec((1, D), lambda: (0, 0)),
            pl.BlockSpec((D, NCLS), lambda: (0, 0)),
            pl.BlockSpec((1, NCLS), lambda: (0, 0)),
        ],
        out_specs=pl.BlockSpec((NG, NCLS), lambda: (0, 0)),
        out_shape=jax.ShapeDtypeStruct((NG, NCLS), jnp.float32),
    )(h1, x, batch.astype(jnp.int32).reshape(N, 1),
      jnp.concatenate([jnp.full((1,), -1, jnp.int32),
                       batch.astype(jnp.int32)[:-1]]).reshape(N, 1),
      W0, b0.reshape(1, D), Wl1, bl1.reshape(1, D), W2, b2.reshape(1, NCLS))
    return out
